# baseline (speedup 1.0000x reference)
import jax
import jax.numpy as jnp
from jax.experimental import pallas as pl

_TILE = 512


def _moe_kernel(x_ref, w1_ref, b1_ref, w2_ref, b2_ref, wg_ref, bg_ref, o_ref):
    E, _, H = w1_ref.shape
    x = x_ref[...]

    logits = jax.lax.dot_general(
        x, wg_ref[...], (((1,), (0,)), ((), ()))) + bg_ref[...]
    m = jnp.max(logits, axis=1, keepdims=True)
    ex = jnp.exp(logits - m)
    probs = ex / jnp.sum(ex, axis=1, keepdims=True)

    iota = jax.lax.broadcasted_iota(jnp.int32, probs.shape, 1)
    m1 = jnp.max(probs, axis=1, keepdims=True)
    i1 = jnp.min(jnp.where(probs == m1, iota, 127), axis=1, keepdims=True)
    sel1 = iota == i1
    p2 = jnp.where(sel1, -1.0, probs)
    m2 = jnp.max(p2, axis=1, keepdims=True)
    i2 = jnp.min(jnp.where(p2 == m2, iota, 127), axis=1, keepdims=True)
    sel2 = iota == i2
    w = jnp.where(sel1, m1, 0.0) + jnp.where(sel2, m2, 0.0)

    xb = x.astype(jnp.bfloat16)
    hs_parts = []
    for e in range(E):
        h = jnp.maximum(
            jax.lax.dot_general(
                xb, w1_ref[e], (((1,), (0,)), ((), ())),
                preferred_element_type=jnp.float32) + b1_ref[e:e + 1, :],
            0.0)
        hs_parts.append((h * w[:, e:e + 1]).astype(jnp.bfloat16))
    hs = jnp.concatenate(hs_parts, axis=1)
    acc = jax.lax.dot_general(
        hs, w2_ref[...], (((1,), (0,)), ((), ())),
        preferred_element_type=jnp.float32)
    acc = acc + jax.lax.dot_general(
        w.astype(jnp.bfloat16), b2_ref[...],
        (((1,), (0,)), ((), ())), preferred_element_type=jnp.float32)
    o_ref[...] = acc


def kernel(x, W1, b1, W2, b2, Wg, bg):
    B, S, D = x.shape
    E, _, H = W1.shape
    O = W2.shape[2]
    N = B * S
    xf = x.reshape(N, D)
    W1b = W1.astype(jnp.bfloat16)
    W2b = W2.astype(jnp.bfloat16).reshape(E * H, O)
    b2b = b2.astype(jnp.bfloat16)
    bgr = bg.reshape(1, E)
    out = pl.pallas_call(
        _moe_kernel,
        grid=(N // _TILE,),
        in_specs=[
            pl.BlockSpec((_TILE, D), lambda i: (i, 0)),
            pl.BlockSpec((E, D, H), lambda i: (0, 0, 0)),
            pl.BlockSpec((E, H), lambda i: (0, 0)),
            pl.BlockSpec((E * H, O), lambda i: (0, 0)),
            pl.BlockSpec((E, O), lambda i: (0, 0)),
            pl.BlockSpec((D, E), lambda i: (0, 0)),
            pl.BlockSpec((1, E), lambda i: (0, 0)),
        ],
        out_specs=pl.BlockSpec((_TILE, O), lambda i: (i, 0)),
        out_shape=jax.ShapeDtypeStruct((N, O), jnp.float32),
    )(xf, W1b, b1, W2b, b2b, Wg, bgr)
    return out.reshape(B, S, O)


# speedup vs baseline: 1.0751x; 1.0751x over previous
import jax
import jax.numpy as jnp
from jax.experimental import pallas as pl

_TILE = 512


def _moe_kernel(x_ref, w1_ref, b1_ref, w2_ref, b2_ref, wg_ref, bg_ref, o_ref):
    E, _, H = w1_ref.shape
    x = x_ref[...]

    logits = jax.lax.dot_general(
        x, wg_ref[...], (((1,), (0,)), ((), ()))) + bg_ref[...]
    m = jnp.max(logits, axis=1, keepdims=True)
    ex = jnp.exp(logits - m)
    probs = ex / jnp.sum(ex, axis=1, keepdims=True)

    iota = jax.lax.broadcasted_iota(jnp.int32, probs.shape, 1)
    m1 = jnp.max(probs, axis=1, keepdims=True)
    i1 = jnp.min(jnp.where(probs == m1, iota, 127), axis=1, keepdims=True)
    sel1 = iota == i1
    p2 = jnp.where(sel1, -1.0, probs)
    m2 = jnp.max(p2, axis=1, keepdims=True)
    i2 = jnp.min(jnp.where(p2 == m2, iota, 127), axis=1, keepdims=True)
    sel2 = iota == i2
    w = jnp.where(sel1, m1, 0.0) + jnp.where(sel2, m2, 0.0)

    xb = x.astype(jnp.bfloat16)
    wb = w.astype(jnp.bfloat16)
    acc = jax.lax.dot_general(
        wb, b2_ref[...],
        (((1,), (0,)), ((), ())), preferred_element_type=jnp.float32)
    for e in range(E):
        h = jnp.maximum(
            jax.lax.dot_general(
                xb, w1_ref[e], (((1,), (0,)), ((), ())),
                preferred_element_type=jnp.float32) + b1_ref[e:e + 1, :],
            0.0)
        hs = h.astype(jnp.bfloat16) * wb[:, e:e + 1]
        acc = acc + jax.lax.dot_general(
            hs, w2_ref[e], (((1,), (0,)), ((), ())),
            preferred_element_type=jnp.float32)
    o_ref[...] = acc


def kernel(x, W1, b1, W2, b2, Wg, bg):
    B, S, D = x.shape
    E, _, H = W1.shape
    O = W2.shape[2]
    N = B * S
    xf = x.reshape(N, D)
    W1b = W1.astype(jnp.bfloat16)
    W2b = W2.astype(jnp.bfloat16)
    b2b = b2.astype(jnp.bfloat16)
    bgr = bg.reshape(1, E)
    out = pl.pallas_call(
        _moe_kernel,
        grid=(N // _TILE,),
        in_specs=[
            pl.BlockSpec((_TILE, D), lambda i: (i, 0)),
            pl.BlockSpec((E, D, H), lambda i: (0, 0, 0)),
            pl.BlockSpec((E, H), lambda i: (0, 0)),
            pl.BlockSpec((E, H, O), lambda i: (0, 0, 0)),
            pl.BlockSpec((E, O), lambda i: (0, 0)),
            pl.BlockSpec((D, E), lambda i: (0, 0)),
            pl.BlockSpec((1, E), lambda i: (0, 0)),
        ],
        out_specs=pl.BlockSpec((_TILE, O), lambda i: (i, 0)),
        out_shape=jax.ShapeDtypeStruct((N, O), jnp.float32),
    )(xf, W1b, b1, W2b, b2b, Wg, bgr)
    return out.reshape(B, S, O)


# speedup vs baseline: 1.1603x; 1.0793x over previous
import jax
import jax.numpy as jnp
from jax.experimental import pallas as pl

_TILE = 1024


def _moe_kernel(x_ref, w1_ref, b1_ref, w2_ref, b2_ref, wg_ref, bg_ref, o_ref):
    E, _, H = w1_ref.shape
    x = x_ref[...]

    logits = jax.lax.dot_general(
        x, wg_ref[...], (((1,), (0,)), ((), ()))) + bg_ref[...]
    m = jnp.max(logits, axis=1, keepdims=True)
    ex = jnp.exp(logits - m)
    probs = ex / jnp.sum(ex, axis=1, keepdims=True)

    iota = jax.lax.broadcasted_iota(jnp.int32, probs.shape, 1)
    m1 = jnp.max(probs, axis=1, keepdims=True)
    i1 = jnp.min(jnp.where(probs == m1, iota, 127), axis=1, keepdims=True)
    sel1 = iota == i1
    p2 = jnp.where(sel1, -1.0, probs)
    m2 = jnp.max(p2, axis=1, keepdims=True)
    i2 = jnp.min(jnp.where(p2 == m2, iota, 127), axis=1, keepdims=True)
    sel2 = iota == i2
    w = jnp.where(sel1, m1, 0.0) + jnp.where(sel2, m2, 0.0)

    xb = x.astype(jnp.bfloat16)
    wb = w.astype(jnp.bfloat16)
    acc = jax.lax.dot_general(
        wb, b2_ref[...],
        (((1,), (0,)), ((), ())), preferred_element_type=jnp.float32)
    for e in range(E):
        h = jnp.maximum(
            jax.lax.dot_general(
                xb, w1_ref[e], (((1,), (0,)), ((), ())),
                preferred_element_type=jnp.float32) + b1_ref[e:e + 1, :],
            0.0)
        hs = h.astype(jnp.bfloat16) * wb[:, e:e + 1]
        acc = acc + jax.lax.dot_general(
            hs, w2_ref[e], (((1,), (0,)), ((), ())),
            preferred_element_type=jnp.float32)
    o_ref[...] = acc


def kernel(x, W1, b1, W2, b2, Wg, bg):
    B, S, D = x.shape
    E, _, H = W1.shape
    O = W2.shape[2]
    N = B * S
    xf = x.reshape(N, D)
    W1b = W1.astype(jnp.bfloat16)
    W2b = W2.astype(jnp.bfloat16)
    b2b = b2.astype(jnp.bfloat16)
    bgr = bg.reshape(1, E)
    out = pl.pallas_call(
        _moe_kernel,
        grid=(N // _TILE,),
        in_specs=[
            pl.BlockSpec((_TILE, D), lambda i: (i, 0)),
            pl.BlockSpec((E, D, H), lambda i: (0, 0, 0)),
            pl.BlockSpec((E, H), lambda i: (0, 0)),
            pl.BlockSpec((E, H, O), lambda i: (0, 0, 0)),
            pl.BlockSpec((E, O), lambda i: (0, 0)),
            pl.BlockSpec((D, E), lambda i: (0, 0)),
            pl.BlockSpec((1, E), lambda i: (0, 0)),
        ],
        out_specs=pl.BlockSpec((_TILE, O), lambda i: (i, 0)),
        out_shape=jax.ShapeDtypeStruct((N, O), jnp.float32),
    )(xf, W1b, b1, W2b, b2b, Wg, bgr)
    return out.reshape(B, S, O)


# speedup vs baseline: 1.1686x; 1.0071x over previous
import jax
import jax.numpy as jnp
from jax.experimental import pallas as pl

_TILE = 2048


def _moe_kernel(x_ref, w1_ref, b1_ref, w2_ref, b2_ref, wg_ref, bg_ref, o_ref):
    E, _, H = w1_ref.shape
    x = x_ref[...]

    logits = jax.lax.dot_general(
        x, wg_ref[...], (((1,), (0,)), ((), ()))) + bg_ref[...]
    m = jnp.max(logits, axis=1, keepdims=True)
    ex = jnp.exp(logits - m)
    probs = ex / jnp.sum(ex, axis=1, keepdims=True)

    iota = jax.lax.broadcasted_iota(jnp.int32, probs.shape, 1)
    m1 = jnp.max(probs, axis=1, keepdims=True)
    i1 = jnp.min(jnp.where(probs == m1, iota, 127), axis=1, keepdims=True)
    sel1 = iota == i1
    p2 = jnp.where(sel1, -1.0, probs)
    m2 = jnp.max(p2, axis=1, keepdims=True)
    i2 = jnp.min(jnp.where(p2 == m2, iota, 127), axis=1, keepdims=True)
    sel2 = iota == i2
    w = jnp.where(sel1, m1, 0.0) + jnp.where(sel2, m2, 0.0)

    xb = x.astype(jnp.bfloat16)
    wb = w.astype(jnp.bfloat16)
    acc = jax.lax.dot_general(
        wb, b2_ref[...],
        (((1,), (0,)), ((), ())), preferred_element_type=jnp.float32)
    for e in range(E):
        h = jnp.maximum(
            jax.lax.dot_general(
                xb, w1_ref[e], (((1,), (0,)), ((), ())),
                preferred_element_type=jnp.float32) + b1_ref[e:e + 1, :],
            0.0)
        hs = h.astype(jnp.bfloat16) * wb[:, e:e + 1]
        acc = acc + jax.lax.dot_general(
            hs, w2_ref[e], (((1,), (0,)), ((), ())),
            preferred_element_type=jnp.float32)
    o_ref[...] = acc


def kernel(x, W1, b1, W2, b2, Wg, bg):
    B, S, D = x.shape
    E, _, H = W1.shape
    O = W2.shape[2]
    N = B * S
    xf = x.reshape(N, D)
    W1b = W1.astype(jnp.bfloat16)
    W2b = W2.astype(jnp.bfloat16)
    b2b = b2.astype(jnp.bfloat16)
    bgr = bg.reshape(1, E)
    out = pl.pallas_call(
        _moe_kernel,
        grid=(N // _TILE,),
        in_specs=[
            pl.BlockSpec((_TILE, D), lambda i: (i, 0)),
            pl.BlockSpec((E, D, H), lambda i: (0, 0, 0)),
            pl.BlockSpec((E, H), lambda i: (0, 0)),
            pl.BlockSpec((E, H, O), lambda i: (0, 0, 0)),
            pl.BlockSpec((E, O), lambda i: (0, 0)),
            pl.BlockSpec((D, E), lambda i: (0, 0)),
            pl.BlockSpec((1, E), lambda i: (0, 0)),
        ],
        out_specs=pl.BlockSpec((_TILE, O), lambda i: (i, 0)),
        out_shape=jax.ShapeDtypeStruct((N, O), jnp.float32),
    )(xf, W1b, b1, W2b, b2b, Wg, bgr)
    return out.reshape(B, S, O)


# speedup vs baseline: 1.2405x; 1.0615x over previous
import jax
import jax.numpy as jnp
from jax.experimental import pallas as pl

_TILE = 1024


def _moe_kernel(x_ref, w1_ref, b1_ref, w2_ref, b2_ref, wg_ref, bg_ref, o_ref):
    E, _, H = w1_ref.shape
    x = x_ref[...]

    logits = jax.lax.dot_general(
        x, wg_ref[...], (((1,), (0,)), ((), ()))) + bg_ref[...]
    m = jnp.max(logits, axis=1, keepdims=True)
    ex = jnp.exp(logits - m)
    probs = ex / jnp.sum(ex, axis=1, keepdims=True)

    iota = jax.lax.broadcasted_iota(jnp.int32, probs.shape, 1)
    m1 = jnp.max(probs, axis=1, keepdims=True)
    i1 = jnp.min(jnp.where(probs == m1, iota, 127), axis=1, keepdims=True)
    sel1 = iota == i1
    p2 = jnp.where(sel1, -1.0, probs)
    m2 = jnp.max(p2, axis=1, keepdims=True)
    i2 = jnp.min(jnp.where(p2 == m2, iota, 127), axis=1, keepdims=True)
    sel2 = iota == i2
    w = jnp.where(sel1, m1, 0.0) + jnp.where(sel2, m2, 0.0)

    xb = x.astype(jnp.bfloat16)
    wb = w.astype(jnp.bfloat16)
    hs = []
    for e in range(E):
        h = jnp.maximum(
            jax.lax.dot_general(
                xb, w1_ref[e], (((1,), (0,)), ((), ())),
                preferred_element_type=jnp.float32) + b1_ref[e:e + 1, :],
            0.0)
        hs.append(h.astype(jnp.bfloat16) * wb[:, e:e + 1])
    acc = jax.lax.dot_general(
        wb, b2_ref[...],
        (((1,), (0,)), ((), ())), preferred_element_type=jnp.float32)
    for e in range(E):
        acc = acc + jax.lax.dot_general(
            hs[e], w2_ref[e], (((1,), (0,)), ((), ())),
            preferred_element_type=jnp.float32)
    o_ref[...] = acc


def kernel(x, W1, b1, W2, b2, Wg, bg):
    B, S, D = x.shape
    E, _, H = W1.shape
    O = W2.shape[2]
    N = B * S
    xf = x.reshape(N, D)
    W1b = W1.astype(jnp.bfloat16)
    W2b = W2.astype(jnp.bfloat16)
    b2b = b2.astype(jnp.bfloat16)
    bgr = bg.reshape(1, E)
    out = pl.pallas_call(
        _moe_kernel,
        grid=(N // _TILE,),
        in_specs=[
            pl.BlockSpec((_TILE, D), lambda i: (i, 0)),
            pl.BlockSpec((E, D, H), lambda i: (0, 0, 0)),
            pl.BlockSpec((E, H), lambda i: (0, 0)),
            pl.BlockSpec((E, H, O), lambda i: (0, 0, 0)),
            pl.BlockSpec((E, O), lambda i: (0, 0)),
            pl.BlockSpec((D, E), lambda i: (0, 0)),
            pl.BlockSpec((1, E), lambda i: (0, 0)),
        ],
        out_specs=pl.BlockSpec((_TILE, O), lambda i: (i, 0)),
        out_shape=jax.ShapeDtypeStruct((N, O), jnp.float32),
    )(xf, W1b, b1, W2b, b2b, Wg, bgr)
    return out.reshape(B, S, O)


# speedup vs baseline: 1.3814x; 1.1136x over previous
import jax
import jax.numpy as jnp
from jax.experimental import pallas as pl

_TILE = 1024


def _moe_kernel(x_ref, w1_ref, b1_ref, w2_ref, b2_ref, wg_ref, bg_ref, o_ref):
    E, _, H = w1_ref.shape
    x = x_ref[...]

    logits = jax.lax.dot_general(
        x, wg_ref[...], (((1,), (0,)), ((), ()))) + bg_ref[...]
    m = jnp.max(logits, axis=1, keepdims=True)
    ex = jnp.exp(logits - m)
    probs = ex / jnp.sum(ex, axis=1, keepdims=True)

    iota = jax.lax.broadcasted_iota(jnp.int32, probs.shape, 1)
    m1 = jnp.max(probs, axis=1, keepdims=True)
    i1 = jnp.min(jnp.where(probs == m1, iota, 127), axis=1, keepdims=True)
    sel1 = iota == i1
    p2 = jnp.where(sel1, -1.0, probs)
    m2 = jnp.max(p2, axis=1, keepdims=True)
    i2 = jnp.min(jnp.where(p2 == m2, iota, 127), axis=1, keepdims=True)
    sel2 = iota == i2
    w = jnp.where(sel1, m1, 0.0) + jnp.where(sel2, m2, 0.0)

    xb = x.astype(jnp.bfloat16)
    wb = w.astype(jnp.bfloat16)
    hs = []
    for e in range(E):
        h = jnp.maximum(
            jax.lax.dot_general(
                xb, w1_ref[e].astype(jnp.bfloat16), (((1,), (0,)), ((), ())),
                preferred_element_type=jnp.float32) + b1_ref[e:e + 1, :],
            0.0)
        hs.append(h.astype(jnp.bfloat16) * wb[:, e:e + 1])
    acc = jax.lax.dot_general(
        wb, b2_ref[...],
        (((1,), (0,)), ((), ())), preferred_element_type=jnp.float32)
    for e in range(E):
        acc = acc + jax.lax.dot_general(
            hs[e], w2_ref[e].astype(jnp.bfloat16), (((1,), (0,)), ((), ())),
            preferred_element_type=jnp.float32)
    o_ref[...] = acc


def kernel(x, W1, b1, W2, b2, Wg, bg):
    B, S, D = x.shape
    E, _, H = W1.shape
    O = W2.shape[2]
    N = B * S
    xf = x.reshape(N, D)
    b2b = b2.astype(jnp.bfloat16)
    bgr = bg.reshape(1, E)
    out = pl.pallas_call(
        _moe_kernel,
        grid=(N // _TILE,),
        in_specs=[
            pl.BlockSpec((_TILE, D), lambda i: (i, 0)),
            pl.BlockSpec((E, D, H), lambda i: (0, 0, 0)),
            pl.BlockSpec((E, H), lambda i: (0, 0)),
            pl.BlockSpec((E, H, O), lambda i: (0, 0, 0)),
            pl.BlockSpec((E, O), lambda i: (0, 0)),
            pl.BlockSpec((D, E), lambda i: (0, 0)),
            pl.BlockSpec((1, E), lambda i: (0, 0)),
        ],
        out_specs=pl.BlockSpec((_TILE, O), lambda i: (i, 0)),
        out_shape=jax.ShapeDtypeStruct((N, O), jnp.float32),
    )(xf, W1, b1, W2, b2b, Wg, bgr)
    return out.reshape(B, S, O)
